# baseline (speedup 1.0000x reference)
import functools

import jax
import jax.numpy as jnp
from jax import lax
from jax.experimental import pallas as pl
from jax.experimental.pallas import tpu as pltpu
from jax.experimental.pallas import tpu_sc as plsc

DIM = 768
RANK = 16
WINDOW = 64
HALF = 64
SUB = 128
WIN = 2 * SUB
EPS = 1e-5


def _ln(x, w, b):
    mu = jnp.mean(x, axis=-1, keepdims=True)
    var = jnp.mean((x - mu) ** 2, axis=-1, keepdims=True)
    return (x - mu) * lax.rsqrt(var + EPS) * w + b


def _sc_gather(table, ids_flat):
    info = plsc.get_sparse_core_info()
    nw = info.num_cores * info.num_subcores
    n = ids_flat.shape[0]
    per_w = n // nw
    mesh = plsc.VectorSubcoreMesh(core_axis_name="c", subcore_axis_name="s")

    @functools.partial(
        pl.kernel,
        mesh=mesh,
        out_type=jax.ShapeDtypeStruct((n, DIM), jnp.float32),
        scratch_types=[
            pltpu.VMEM((per_w,), jnp.int32),
            pltpu.VMEM((per_w, DIM), jnp.float32),
            pltpu.SemaphoreType.DMA,
        ],
    )
    def k(table_hbm, idx_hbm, out_hbm, idx_v, rows_v, sem):
        wid = lax.axis_index("s") * info.num_cores + lax.axis_index("c")
        base = wid * per_w
        pltpu.sync_copy(idx_hbm.at[pl.ds(base, per_w)], idx_v)
        pltpu.async_copy(table_hbm.at[idx_v], rows_v, sem).wait()
        pltpu.sync_copy(rows_v, out_hbm.at[pl.ds(base, per_w)])

    return k(table, ids_flat)


def _fused_body(emb_ref, pos_ref, anc_ref, u_ref, v_ref, nw_ref, nb_ref,
                inw_ref, inb_ref, out_ref):
    s = pos_ref.shape[0]
    nlayers = u_ref.shape[0]

    x = _ln(emb_ref[0] + pos_ref[...], inw_ref[...], inb_ref[...])
    a_row = anc_ref[...]

    for l in range(nlayers):
        u = u_ref[l]
        v = v_ref[l]
        nw = nw_ref[l:l + 1]
        nb = nb_ref[l:l + 1]

        xb = x.astype(jnp.bfloat16)
        q = jnp.dot(x, u, preferred_element_type=jnp.float32)
        kk = jnp.dot(x, v, preferred_element_type=jnp.float32)
        q0 = jnp.dot(a_row, u, preferred_element_type=jnp.float32)
        k0 = jnp.dot(a_row, v, preferred_element_type=jnp.float32)

        parts = []
        for t in range(s // SUB):
            off = SUB * t
            ws = min(max(off - HALF, 0), s - WIN)
            q_t = q[off:off + SUB]
            k_t = kk[ws:ws + WIN]
            v_t = xb[ws:ws + WIN]
            scores = lax.dot_general(q_t, k_t, (((1,), (1,)), ((), ())),
                                     preferred_element_type=jnp.float32) * 0.25
            s0 = lax.dot_general(q_t, k0, (((1,), (1,)), ((), ())),
                                 preferred_element_type=jnp.float32) * 0.25
            io = lax.broadcasted_iota(jnp.int32, (SUB, WIN), 0)
            jo = lax.broadcasted_iota(jnp.int32, (SUB, WIN), 1)
            valid = jnp.abs(io - jo + (off - ws)) <= WINDOW

            asc = jnp.abs(scores)
            m_row = jnp.maximum(
                jnp.max(jnp.where(valid, asc, -jnp.inf), axis=1, keepdims=True),
                jnp.abs(s0))
            e = jnp.where(valid, jnp.exp(asc - m_row), 0.0)
            e0 = jnp.exp(jnp.abs(s0) - m_row)
            denom = jnp.sum(e, axis=1, keepdims=True) + e0
            wgt = jnp.sign(scores) * (e / denom)
            w0 = jnp.sign(s0) * (e0 / denom)

            delta = jnp.dot(wgt.astype(jnp.bfloat16), v_t,
                            preferred_element_type=jnp.float32) + w0 * a_row
            parts.append(_ln(x[off:off + SUB] + delta, nw, nb))

        s_all = lax.dot_general(q0, kk, (((1,), (1,)), ((), ())),
                                preferred_element_type=jnp.float32) * 0.25
        s00 = lax.dot_general(q0, k0, (((1,), (1,)), ((), ())),
                              preferred_element_type=jnp.float32) * 0.25
        m0 = jnp.maximum(jnp.max(jnp.abs(s_all)), jnp.abs(s00)[0, 0])
        e_all = jnp.exp(jnp.abs(s_all) - m0)
        e00 = jnp.exp(jnp.abs(s00) - m0)
        den0 = jnp.sum(e_all) + e00[0, 0]
        delta0 = (jnp.dot(jnp.sign(s_all) * e_all, x,
                          preferred_element_type=jnp.float32)
                  + jnp.sign(s00) * e00 * a_row) / den0
        a_row = _ln(a_row + delta0, nw, nb)

        x = jnp.concatenate(parts, axis=0)

    shifted = jnp.concatenate([a_row, x[:s - 1]], axis=0)
    out_ref[0, 0:s, :] = shifted
    out_ref[0, s:s + 1, :] = x[s - 1:]


def _fused(emb, pos, anchor_row, u, v, nw, nb, inw, inb):
    b, s, _ = emb.shape
    return pl.pallas_call(
        _fused_body,
        grid=(b,),
        in_specs=[
            pl.BlockSpec((1, s, DIM), lambda bi: (bi, 0, 0)),
            pl.BlockSpec((s, DIM), lambda bi: (0, 0)),
            pl.BlockSpec((1, DIM), lambda bi: (0, 0)),
            pl.BlockSpec(u.shape, lambda bi: (0, 0, 0)),
            pl.BlockSpec(v.shape, lambda bi: (0, 0, 0)),
            pl.BlockSpec(nw.shape, lambda bi: (0, 0)),
            pl.BlockSpec(nb.shape, lambda bi: (0, 0)),
            pl.BlockSpec((1, DIM), lambda bi: (0, 0)),
            pl.BlockSpec((1, DIM), lambda bi: (0, 0)),
        ],
        out_specs=pl.BlockSpec((1, s + 1, DIM), lambda bi: (bi, 0, 0)),
        out_shape=jax.ShapeDtypeStruct((b, s + 1, DIM), jnp.float32),
        compiler_params=pltpu.CompilerParams(
            dimension_semantics=("arbitrary",)),
    )(emb, pos, anchor_row, u, v, nw, nb, inw, inb)


def kernel(input_ids, tok_emb, pos_emb, anchor_state, anchor_val, in_norm_w,
           in_norm_b, sp_w, sp_b, U, V, norm_w, norm_b):
    del anchor_state, sp_w, sp_b
    bsz, seq = input_ids.shape
    ids = input_ids.reshape(-1).astype(jnp.int32)
    emb = _sc_gather(tok_emb, ids).reshape(bsz, seq, DIM)
    return _fused(emb, pos_emb[:seq], anchor_val.reshape(1, DIM),
                  U[:, 0], V[:, 0], norm_w, norm_b,
                  in_norm_w.reshape(1, DIM), in_norm_b.reshape(1, DIM))


# speedup vs baseline: 1.0008x; 1.0008x over previous
import functools

import jax
import jax.numpy as jnp
from jax import lax
from jax.experimental import pallas as pl
from jax.experimental.pallas import tpu as pltpu
from jax.experimental.pallas import tpu_sc as plsc

DIM = 768
RANK = 16
WINDOW = 64
HALF = 64
SUB = 128
WIN = 2 * SUB
EPS = 1e-5


def _ln(x, w, b):
    mu = jnp.mean(x, axis=-1, keepdims=True)
    var = jnp.mean((x - mu) ** 2, axis=-1, keepdims=True)
    return (x - mu) * lax.rsqrt(var + EPS) * w + b


def _sc_gather(table, ids_flat):
    info = plsc.get_sparse_core_info()
    nw = info.num_cores * info.num_subcores
    n = ids_flat.shape[0]
    per_w = n // nw
    mesh = plsc.VectorSubcoreMesh(core_axis_name="c", subcore_axis_name="s")

    @functools.partial(
        pl.kernel,
        mesh=mesh,
        out_type=jax.ShapeDtypeStruct((n, DIM), jnp.float32),
        scratch_types=[
            pltpu.VMEM((per_w,), jnp.int32),
            pltpu.VMEM((per_w, DIM), jnp.float32),
            pltpu.SemaphoreType.DMA,
        ],
    )
    def k(table_hbm, idx_hbm, out_hbm, idx_v, rows_v, sem):
        wid = lax.axis_index("s") * info.num_cores + lax.axis_index("c")
        base = wid * per_w
        pltpu.sync_copy(idx_hbm.at[pl.ds(base, per_w)], idx_v)
        pltpu.async_copy(table_hbm.at[idx_v], rows_v, sem).wait()
        pltpu.sync_copy(rows_v, out_hbm.at[pl.ds(base, per_w)])

    return k(table, ids_flat)


def _fused_body(emb_ref, pos_ref, anc_ref, u_ref, v_ref, nw_ref, nb_ref,
                inw_ref, inb_ref, out_ref):
    s = pos_ref.shape[0]
    nlayers = u_ref.shape[0]

    x = _ln(emb_ref[0] + pos_ref[...], inw_ref[...], inb_ref[...])
    a_row = anc_ref[...]

    for l in range(nlayers):
        u = u_ref[l]
        v = v_ref[l]
        nw = nw_ref[l:l + 1]
        nb = nb_ref[l:l + 1]

        uv = jnp.concatenate([u, v], axis=1)
        qk = jnp.dot(x, uv, preferred_element_type=jnp.float32)
        q = qk[:, :RANK]
        kk = qk[:, RANK:]
        qk0 = jnp.dot(a_row, uv, preferred_element_type=jnp.float32)
        q0 = qk0[:, :RANK]
        k0 = qk0[:, RANK:]

        parts = []
        for t in range(s // SUB):
            off = SUB * t
            ws = min(max(off - HALF, 0), s - WIN)
            q_t = q[off:off + SUB]
            k_t = kk[ws:ws + WIN]
            v_t = x[ws:ws + WIN]
            scores = lax.dot_general(q_t, k_t, (((1,), (1,)), ((), ())),
                                     preferred_element_type=jnp.float32) * 0.25
            s0 = lax.dot_general(q_t, k0, (((1,), (1,)), ((), ())),
                                 preferred_element_type=jnp.float32) * 0.25
            io = lax.broadcasted_iota(jnp.int32, (SUB, WIN), 0)
            jo = lax.broadcasted_iota(jnp.int32, (SUB, WIN), 1)
            valid = jnp.abs(io - jo + (off - ws)) <= WINDOW

            asc = jnp.abs(scores)
            m_row = jnp.maximum(
                jnp.max(jnp.where(valid, asc, -jnp.inf), axis=1, keepdims=True),
                jnp.abs(s0))
            e = jnp.where(valid, jnp.exp(asc - m_row), 0.0)
            e0 = jnp.exp(jnp.abs(s0) - m_row)
            rden = 1.0 / (jnp.sum(e, axis=1, keepdims=True) + e0)
            wgt = jnp.sign(scores) * (e * rden)
            w0 = jnp.sign(s0) * (e0 * rden)

            delta = jnp.dot(wgt, v_t,
                            preferred_element_type=jnp.float32) + w0 * a_row
            parts.append(_ln(x[off:off + SUB] + delta, nw, nb))

        s_all = lax.dot_general(q0, kk, (((1,), (1,)), ((), ())),
                                preferred_element_type=jnp.float32) * 0.25
        s00 = lax.dot_general(q0, k0, (((1,), (1,)), ((), ())),
                              preferred_element_type=jnp.float32) * 0.25
        m0 = jnp.maximum(jnp.max(jnp.abs(s_all)), jnp.abs(s00)[0, 0])
        e_all = jnp.exp(jnp.abs(s_all) - m0)
        e00 = jnp.exp(jnp.abs(s00) - m0)
        den0 = jnp.sum(e_all) + e00[0, 0]
        delta0 = (jnp.dot(jnp.sign(s_all) * e_all, x,
                          preferred_element_type=jnp.float32)
                  + jnp.sign(s00) * e00 * a_row) / den0
        a_row = _ln(a_row + delta0, nw, nb)

        x = jnp.concatenate(parts, axis=0)

    shifted = jnp.concatenate([a_row, x[:s - 1]], axis=0)
    out_ref[0, 0:s, :] = shifted
    out_ref[0, s:s + 1, :] = x[s - 1:]


def _fused(emb, pos, anchor_row, u, v, nw, nb, inw, inb):
    b, s, _ = emb.shape
    return pl.pallas_call(
        _fused_body,
        grid=(b,),
        in_specs=[
            pl.BlockSpec((1, s, DIM), lambda bi: (bi, 0, 0)),
            pl.BlockSpec((s, DIM), lambda bi: (0, 0)),
            pl.BlockSpec((1, DIM), lambda bi: (0, 0)),
            pl.BlockSpec(u.shape, lambda bi: (0, 0, 0)),
            pl.BlockSpec(v.shape, lambda bi: (0, 0, 0)),
            pl.BlockSpec(nw.shape, lambda bi: (0, 0)),
            pl.BlockSpec(nb.shape, lambda bi: (0, 0)),
            pl.BlockSpec((1, DIM), lambda bi: (0, 0)),
            pl.BlockSpec((1, DIM), lambda bi: (0, 0)),
        ],
        out_specs=pl.BlockSpec((1, s + 1, DIM), lambda bi: (bi, 0, 0)),
        out_shape=jax.ShapeDtypeStruct((b, s + 1, DIM), jnp.float32),
        compiler_params=pltpu.CompilerParams(
            dimension_semantics=("arbitrary",)),
    )(emb, pos, anchor_row, u, v, nw, nb, inw, inb)


def kernel(input_ids, tok_emb, pos_emb, anchor_state, anchor_val, in_norm_w,
           in_norm_b, sp_w, sp_b, U, V, norm_w, norm_b):
    del anchor_state, sp_w, sp_b
    bsz, seq = input_ids.shape
    ids = input_ids.reshape(-1).astype(jnp.int32)
    emb = _sc_gather(tok_emb, ids).reshape(bsz, seq, DIM)
    return _fused(emb, pos_emb[:seq], anchor_val.reshape(1, DIM),
                  U[:, 0], V[:, 0], norm_w, norm_b,
                  in_norm_w.reshape(1, DIM), in_norm_b.reshape(1, DIM))


# speedup vs baseline: 1.0031x; 1.0023x over previous
import functools

import jax
import jax.numpy as jnp
from jax import lax
from jax.experimental import pallas as pl
from jax.experimental.pallas import tpu as pltpu
from jax.experimental.pallas import tpu_sc as plsc

DIM = 768
RANK = 16
WINDOW = 64
HALF = 64
SUB = 128
WIN = 2 * SUB
EPS = 1e-5


def _ln(x, w, b):
    mu = jnp.mean(x, axis=-1, keepdims=True)
    var = jnp.mean((x - mu) ** 2, axis=-1, keepdims=True)
    return (x - mu) * lax.rsqrt(var + EPS) * w + b


def _sc_gather(table, ids_flat):
    info = plsc.get_sparse_core_info()
    nw = info.num_cores * info.num_subcores
    n = ids_flat.shape[0]
    per_w = n // nw
    mesh = plsc.VectorSubcoreMesh(core_axis_name="c", subcore_axis_name="s")

    @functools.partial(
        pl.kernel,
        mesh=mesh,
        out_type=jax.ShapeDtypeStruct((n, DIM), jnp.float32),
        scratch_types=[
            pltpu.VMEM((per_w,), jnp.int32),
            pltpu.VMEM((per_w, DIM), jnp.float32),
            pltpu.SemaphoreType.DMA,
        ],
    )
    def k(table_hbm, idx_hbm, out_hbm, idx_v, rows_v, sem):
        wid = lax.axis_index("s") * info.num_cores + lax.axis_index("c")
        base = wid * per_w
        pltpu.sync_copy(idx_hbm.at[pl.ds(base, per_w)], idx_v)
        pltpu.async_copy(table_hbm.at[idx_v], rows_v, sem).wait()
        pltpu.sync_copy(rows_v, out_hbm.at[pl.ds(base, per_w)])

    return k(table, ids_flat)


def _fused_body(emb_ref, pos_ref, anc_ref, u_ref, v_ref, nw_ref, nb_ref,
                inw_ref, inb_ref, out_ref):
    s = pos_ref.shape[0]
    nlayers = u_ref.shape[0]

    x = _ln(emb_ref[0] + pos_ref[...], inw_ref[...], inb_ref[...])
    a_row = anc_ref[...]

    for l in range(nlayers):
        u = u_ref[l]
        v = v_ref[l]
        nw = nw_ref[l:l + 1]
        nb = nb_ref[l:l + 1]

        q = jnp.dot(x, u, preferred_element_type=jnp.float32)
        kk = jnp.dot(x, v, preferred_element_type=jnp.float32)
        q0 = jnp.dot(a_row, u, preferred_element_type=jnp.float32)
        k0 = jnp.dot(a_row, v, preferred_element_type=jnp.float32)

        parts = []
        for t in range(s // SUB):
            off = SUB * t
            ws = min(max(off - HALF, 0), s - WIN)
            q_t = q[off:off + SUB]
            k_t = kk[ws:ws + WIN]
            v_t = x[ws:ws + WIN]
            scores = lax.dot_general(q_t, k_t, (((1,), (1,)), ((), ())),
                                     preferred_element_type=jnp.float32) * 0.25
            s0 = lax.dot_general(q_t, k0, (((1,), (1,)), ((), ())),
                                 preferred_element_type=jnp.float32) * 0.25
            io = lax.broadcasted_iota(jnp.int32, (SUB, WIN), 0)
            jo = lax.broadcasted_iota(jnp.int32, (SUB, WIN), 1)
            valid = jnp.abs(io - jo + (off - ws)) <= WINDOW

            asc = jnp.abs(scores)
            m_row = jnp.maximum(
                jnp.max(jnp.where(valid, asc, -jnp.inf), axis=1, keepdims=True),
                jnp.abs(s0))
            e = jnp.where(valid, jnp.exp(asc - m_row), 0.0)
            e0 = jnp.exp(jnp.abs(s0) - m_row)
            denom = jnp.sum(e, axis=1, keepdims=True) + e0
            wgt = jnp.sign(scores) * (e / denom)
            w0 = jnp.sign(s0) * (e0 / denom)

            delta = jnp.dot(wgt, v_t,
                            preferred_element_type=jnp.float32) + w0 * a_row
            parts.append(_ln(x[off:off + SUB] + delta, nw, nb))

        s_all = lax.dot_general(q0, kk, (((1,), (1,)), ((), ())),
                                preferred_element_type=jnp.float32) * 0.25
        s00 = lax.dot_general(q0, k0, (((1,), (1,)), ((), ())),
                              preferred_element_type=jnp.float32) * 0.25
        m0 = jnp.maximum(jnp.max(jnp.abs(s_all)), jnp.abs(s00)[0, 0])
        e_all = jnp.exp(jnp.abs(s_all) - m0)
        e00 = jnp.exp(jnp.abs(s00) - m0)
        den0 = jnp.sum(e_all) + e00[0, 0]
        delta0 = (jnp.dot(jnp.sign(s_all) * e_all, x,
                          preferred_element_type=jnp.float32)
                  + jnp.sign(s00) * e00 * a_row) / den0
        a_row = _ln(a_row + delta0, nw, nb)

        x = jnp.concatenate(parts, axis=0)

    shifted = jnp.concatenate([a_row, x[:s - 1]], axis=0)
    out_ref[0, 0:s, :] = shifted
    out_ref[0, s:s + 1, :] = x[s - 1:]


def _fused(emb, pos, anchor_row, u, v, nw, nb, inw, inb):
    b, s, _ = emb.shape
    return pl.pallas_call(
        _fused_body,
        grid=(b,),
        in_specs=[
            pl.BlockSpec((1, s, DIM), lambda bi: (bi, 0, 0)),
            pl.BlockSpec((s, DIM), lambda bi: (0, 0)),
            pl.BlockSpec((1, DIM), lambda bi: (0, 0)),
            pl.BlockSpec(u.shape, lambda bi: (0, 0, 0)),
            pl.BlockSpec(v.shape, lambda bi: (0, 0, 0)),
            pl.BlockSpec(nw.shape, lambda bi: (0, 0)),
            pl.BlockSpec(nb.shape, lambda bi: (0, 0)),
            pl.BlockSpec((1, DIM), lambda bi: (0, 0)),
            pl.BlockSpec((1, DIM), lambda bi: (0, 0)),
        ],
        out_specs=pl.BlockSpec((1, s + 1, DIM), lambda bi: (bi, 0, 0)),
        out_shape=jax.ShapeDtypeStruct((b, s + 1, DIM), jnp.float32),
        compiler_params=pltpu.CompilerParams(
            dimension_semantics=("arbitrary",)),
    )(emb, pos, anchor_row, u, v, nw, nb, inw, inb)


def kernel(input_ids, tok_emb, pos_emb, anchor_state, anchor_val, in_norm_w,
           in_norm_b, sp_w, sp_b, U, V, norm_w, norm_b):
    del anchor_state, sp_w, sp_b
    bsz, seq = input_ids.shape
    ids = input_ids.reshape(-1).astype(jnp.int32)
    emb = _sc_gather(tok_emb, ids).reshape(bsz, seq, DIM)
    return _fused(emb, pos_emb[:seq], anchor_val.reshape(1, DIM),
                  U[:, 0], V[:, 0], norm_w, norm_b,
                  in_norm_w.reshape(1, DIM), in_norm_b.reshape(1, DIM))
